# baseline (device time: 43896 ns/iter reference)
import jax
import jax.numpy as jnp
from jax import lax
from jax.experimental import pallas as pl
from jax.experimental.pallas import tpu as pltpu

N_DEV = 16
SQ = 256
SKV = 4096
HQ_LOCAL = 8
DH = 128
D_MODEL = 1024
BLK = 64
SCALE = 0.08838834764831843
CHUNK = SQ // N_DEV

_CLS0 = list(range(0, 64, 3))
_L1 = [0, 1] + list(range(2, 64, 3))
_L2 = [0, 2] + list(range(1, 64, 3))
_N0 = len(_CLS0) * BLK
_N1 = len(_L1) * BLK
_N2 = len(_L2) * BLK
_OFF1 = _N0
_OFF2 = _N0 + _N1
_GATHER_ROWS = _N0 + _N1 + _N2

_SEGS = (
    (_CLS0, 0, _N0, (0, 3)),
    (_L1, _OFF1, _N1, (1,)),
    (_L2, _OFF2, _N2, (2,)),
)


def _body(x_ref, wq_ref, k_hbm, v_hbm, wo_ref, out_ref,
          k_buf, v_buf, ctx_buf, p_buf, rs_buf,
          k_sems, v_sems, rs_send_sems, rs_recv_sems,
          ag_send_sems, ag_recv_sems):
    my = lax.axis_index("i")
    head_base = my * HQ_LOCAL

    barrier = pltpu.get_barrier_semaphore()
    for t in range(1, N_DEV):
        pl.semaphore_signal(barrier, inc=1, device_id=((my + t) % N_DEV,),
                            device_id_type=pl.DeviceIdType.MESH)

    copies = [[[] for _ in range(HQ_LOCAL)] for _ in _SEGS]
    for si, (lst, off, _, _) in enumerate(_SEGS):
        for h in range(HQ_LOCAL):
            g = head_base + h
            dst_row = off
            for kb in lst:
                kc = pltpu.make_async_copy(
                    k_hbm.at[0, pl.ds(kb * BLK, BLK), g, :],
                    k_buf.at[h, pl.ds(dst_row, BLK), :],
                    k_sems.at[h])
                vc = pltpu.make_async_copy(
                    v_hbm.at[0, pl.ds(kb * BLK, BLK), g, :],
                    v_buf.at[h, pl.ds(dst_row, BLK), :],
                    v_sems.at[h])
                kc.start()
                vc.start()
                copies[si][h].append((kc, vc))
                dst_row += BLK

    x_b = (x_ref[0] * SCALE).astype(jnp.bfloat16)
    wq_b = wq_ref[...].astype(jnp.bfloat16)
    q = jnp.dot(x_b, wq_b,
                preferred_element_type=jnp.float32).astype(jnp.bfloat16)
    wo_b = wo_ref[...].astype(jnp.bfloat16)

    def attend(q_rows, k_blk, v_blk):
        s = lax.dot_general(q_rows, k_blk.astype(jnp.bfloat16),
                            (((1,), (1,)), ((), ())),
                            preferred_element_type=jnp.float32)
        w = jnp.exp(s.astype(jnp.bfloat16))
        denom = jnp.sum(w, axis=1, keepdims=True, dtype=jnp.float32)
        ctx = jnp.dot(w, v_blk.astype(jnp.bfloat16),
                      preferred_element_type=jnp.float32)
        return ctx * (1.0 / denom)

    barrier_waited = False
    for si, (lst, off, n_rows, qbs) in enumerate(_SEGS):
        for h in range(HQ_LOCAL):
            for kc, vc in copies[si][h]:
                kc.wait()
                vc.wait()
            q_h = q[:, h * DH:(h + 1) * DH]
            cols = slice(h * DH, (h + 1) * DH)
            k_seg = k_buf[h, off:off + n_rows]
            v_seg = v_buf[h, off:off + n_rows]
            if si == 0:
                q03 = jnp.concatenate(
                    [q_h[0:BLK], q_h[3 * BLK:4 * BLK]], axis=0)
                ctx03 = attend(q03, k_seg, v_seg).astype(jnp.bfloat16)
                ctx_buf[0:BLK, cols] = ctx03[0:BLK]
                ctx_buf[3 * BLK:4 * BLK, cols] = ctx03[BLK:2 * BLK]
            else:
                qb = qbs[0]
                ctx_buf[qb * BLK:(qb + 1) * BLK, cols] = attend(
                    q_h[qb * BLK:(qb + 1) * BLK], k_seg, v_seg
                    ).astype(jnp.bfloat16)

        if si == 0:
            ctx_rows = jnp.concatenate(
                [ctx_buf[0:BLK, :], ctx_buf[3 * BLK:4 * BLK, :]], axis=0)
            pr = jnp.dot(ctx_rows, wo_b,
                         preferred_element_type=jnp.float32
                         ).astype(jnp.bfloat16)
            p_buf[0:BLK, :] = pr[0:BLK]
            p_buf[3 * BLK:4 * BLK, :] = pr[BLK:2 * BLK]
        else:
            qb = qbs[0]
            p_buf[qb * BLK:(qb + 1) * BLK, :] = jnp.dot(
                ctx_buf[qb * BLK:(qb + 1) * BLK, :],
                wo_b, preferred_element_type=jnp.float32
                ).astype(jnp.bfloat16)

        if not barrier_waited:
            pl.semaphore_wait(barrier, N_DEV - 1)
            barrier_waited = True

        for t in range(1, N_DEV):
            r = (my + t) % N_DEV
            qb_r = r // (N_DEV // 4)
            cond = (qb_r == qbs[0])
            for extra in qbs[1:]:
                cond = cond | (qb_r == extra)

            @pl.when(cond)
            def _(t=t, r=r):
                pltpu.make_async_remote_copy(
                    src_ref=p_buf.at[pl.ds(r * CHUNK, CHUNK), :],
                    dst_ref=rs_buf.at[t],
                    send_sem=rs_send_sems.at[t],
                    recv_sem=rs_recv_sems.at[t],
                    device_id=(r,),
                    device_id_type=pl.DeviceIdType.MESH,
                ).start()

    for t in range(1, N_DEV):
        pltpu.make_async_remote_copy(
            src_ref=rs_buf.at[t], dst_ref=rs_buf.at[t],
            send_sem=rs_send_sems.at[0], recv_sem=rs_recv_sems.at[t],
            device_id=(my,), device_id_type=pl.DeviceIdType.MESH,
        ).wait_recv()

    own = p_buf[pl.ds(my * CHUNK, CHUNK), :].astype(jnp.float32)
    red = (own + jnp.sum(rs_buf[1:N_DEV].astype(jnp.float32), axis=0)
           ).astype(jnp.bfloat16)
    out_ref[pl.ds(my * CHUNK, CHUNK), :] = red

    ag_sends = []
    for t in range(1, N_DEV):
        rdma = pltpu.make_async_remote_copy(
            src_ref=out_ref.at[pl.ds(my * CHUNK, CHUNK), :],
            dst_ref=out_ref.at[pl.ds(my * CHUNK, CHUNK), :],
            send_sem=ag_send_sems.at[t],
            recv_sem=ag_recv_sems.at[t],
            device_id=((my + t) % N_DEV,),
            device_id_type=pl.DeviceIdType.MESH,
        )
        rdma.start()
        ag_sends.append(rdma)
    for t in range(1, N_DEV):
        pltpu.make_async_remote_copy(
            src_ref=out_ref.at[pl.ds(0, CHUNK), :],
            dst_ref=out_ref.at[pl.ds(0, CHUNK), :],
            send_sem=ag_send_sems.at[0], recv_sem=ag_recv_sems.at[t],
            device_id=(my,), device_id_type=pl.DeviceIdType.MESH,
        ).wait_recv()

    for t in range(1, N_DEV):
        pltpu.make_async_remote_copy(
            src_ref=p_buf.at[pl.ds(0, CHUNK), :],
            dst_ref=rs_buf.at[t],
            send_sem=rs_send_sems.at[t], recv_sem=rs_recv_sems.at[t],
            device_id=(my,), device_id_type=pl.DeviceIdType.MESH,
        ).wait_send()
    for rdma in ag_sends:
        rdma.wait_send()


def kernel(x, Wq, K_ext, V_ext, Wo):
    out = pl.pallas_call(
        _body,
        out_shape=jax.ShapeDtypeStruct((SQ, D_MODEL), jnp.bfloat16),
        in_specs=[
            pl.BlockSpec(memory_space=pltpu.VMEM),
            pl.BlockSpec(memory_space=pltpu.VMEM),
            pl.BlockSpec(memory_space=pl.ANY),
            pl.BlockSpec(memory_space=pl.ANY),
            pl.BlockSpec(memory_space=pltpu.VMEM),
        ],
        out_specs=pl.BlockSpec(memory_space=pltpu.VMEM),
        scratch_shapes=[
            pltpu.VMEM((HQ_LOCAL, _GATHER_ROWS, DH), jnp.float32),
            pltpu.VMEM((HQ_LOCAL, _GATHER_ROWS, DH), jnp.float32),
            pltpu.VMEM((SQ, D_MODEL), jnp.bfloat16),
            pltpu.VMEM((SQ, D_MODEL), jnp.bfloat16),
            pltpu.VMEM((N_DEV, CHUNK, D_MODEL), jnp.bfloat16),
            pltpu.SemaphoreType.DMA((HQ_LOCAL,)),
            pltpu.SemaphoreType.DMA((HQ_LOCAL,)),
            pltpu.SemaphoreType.DMA((N_DEV,)),
            pltpu.SemaphoreType.DMA((N_DEV,)),
            pltpu.SemaphoreType.DMA((N_DEV,)),
            pltpu.SemaphoreType.DMA((N_DEV,)),
        ],
        compiler_params=pltpu.CompilerParams(
            collective_id=0,
            vmem_limit_bytes=100 * 1024 * 1024,
        ),
    )(x, Wq, K_ext, V_ext, Wo)
    return out[None, :, :]
